# baseline (speedup 1.0000x reference)
import functools

import jax
import jax.numpy as jnp
from jax import lax
from jax.experimental import pallas as pl
from jax.experimental.pallas import tpu as pltpu
from jax.experimental.pallas import tpu_sc as plsc

VOCAB = 1000000
EMBED = 64
BATCH = 16384
HIST = 50

NC = 2
NS = 16
NW = NC * NS
L = 16

BB = 128
NBLK = HIST * (BATCH // BB)
BPW = NBLK // NW

_mesh = plsc.VectorSubcoreMesh(core_axis_name="c", subcore_axis_name="s")


@functools.partial(
    pl.kernel,
    mesh=_mesh,
    out_type=jax.ShapeDtypeStruct((HIST, EMBED, BATCH), jnp.float32),
    compiler_params=pltpu.CompilerParams(needs_layout_passes=False),
    scratch_types=[
        pltpu.VMEM((BB,), jnp.int32),
        pltpu.VMEM((BB,), jnp.int32),
        pltpu.VMEM((BB, 128), jnp.float32),
        pltpu.VMEM((EMBED, BB), jnp.float32),
        pltpu.SemaphoreType.DMA,
        pltpu.SemaphoreType.DMA,
    ],
)
def _emb_lookup(x_hbm, a_hbm, out_hbm, idx_v, row_v, pair_v, tile_v, isem, gsem):
    wid = lax.axis_index("s") * NC + lax.axis_index("c")
    blk0 = wid * BPW

    jiota = [lax.iota(jnp.int32, L) + jb * L for jb in range(BB // L)]

    def blk(t, carry):
        b = blk0 + t
        h = b >> 7
        bt = b & 127
        pltpu.sync_copy(x_hbm.at[h, pl.ds(bt * BB, BB)], idx_v)
        colbase = []
        for jb in range(BB // L):
            v = idx_v[pl.ds(jb * L, L)]
            row_v[pl.ds(jb * L, L)] = v >> 1
            colbase.append((v & 1) << 6)
        pltpu.async_copy(a_hbm.at[row_v], pair_v, gsem).wait()
        def erow(e, c):
            for jb in range(BB // L):
                vals = plsc.load_gather(pair_v, [jiota[jb], colbase[jb] + e])
                tile_v[e, pl.ds(jb * L, L)] = vals
            return c
        lax.fori_loop(0, EMBED, erow, 0)
        for et in range(EMBED // 8):
            pltpu.sync_copy(
                tile_v.at[pl.ds(et * 8, 8)],
                out_hbm.at[h, pl.ds(et * 8, 8), pl.ds(bt * BB, BB)],
            )
        return carry

    lax.fori_loop(0, BPW, blk, 0)


def kernel(x, A):
    out = _emb_lookup(x.T, A.reshape(VOCAB // 2, 128))
    return jnp.transpose(out, (2, 0, 1))


# speedup vs baseline: 1.2710x; 1.2710x over previous
import functools

import jax
import jax.numpy as jnp
from jax import lax
from jax.experimental import pallas as pl
from jax.experimental.pallas import tpu as pltpu
from jax.experimental.pallas import tpu_sc as plsc

VOCAB = 1000000
EMBED = 64
BATCH = 16384
HIST = 50

NC = 2
NS = 16
NW = NC * NS
L = 16

BB = 128
PP = 129
NBLK = HIST * (BATCH // BB)
BPW = NBLK // NW

_mesh = plsc.VectorSubcoreMesh(core_axis_name="c", subcore_axis_name="s")


@functools.partial(
    pl.kernel,
    mesh=_mesh,
    out_type=jax.ShapeDtypeStruct((HIST, EMBED, BATCH), jnp.float32),
    compiler_params=pltpu.CompilerParams(needs_layout_passes=False),
    scratch_types=[
        pltpu.VMEM((2, BB), jnp.int32),
        pltpu.VMEM((2, BB), jnp.int32),
        pltpu.VMEM((2, BB), jnp.int32),
        pltpu.VMEM((2, BB, PP), jnp.float32),
        pltpu.VMEM((2, EMBED, BB), jnp.float32),
        pltpu.SemaphoreType.DMA((2,)),
        pltpu.SemaphoreType.DMA((2,)),
        pltpu.SemaphoreType.DMA((2,)),
    ],
)
def _emb_lookup(x_hbm, a_hbm, out_hbm, idx_v, row_v, cb_v, pair_v, tile_v,
                isem, gsem, wsem):
    wid = lax.axis_index("s") * NC + lax.axis_index("c")
    blk0 = wid * BPW

    jiota = [lax.iota(jnp.int32, L) + jb * L for jb in range(BB // L)]

    def start_idx(t, b):
        blk = blk0 + t
        pltpu.async_copy(
            x_hbm.at[blk >> 7, pl.ds((blk & 127) * BB, BB)], idx_v.at[b],
            isem.at[b])

    def wait_idx(b):
        pltpu.make_async_copy(
            x_hbm.at[0, pl.ds(0, BB)], idx_v.at[b], isem.at[b]).wait()

    def gather_dst(b):
        return pair_v.at[b, slice(None), pl.ds(0, BB)]

    def wait_out(b):
        pltpu.make_async_copy(
            tile_v.at[b], out_hbm.at[0, pl.ds(0, EMBED), pl.ds(0, BB)],
            wsem.at[b]).wait()

    start_idx(0, 0)

    def step(t, b):
        bo = 1 - b

        @pl.when(t < BPW)
        def _():
            wait_idx(b)
            for jb in range(BB // L):
                v = idx_v[b, pl.ds(jb * L, L)]
                row_v[b, pl.ds(jb * L, L)] = v >> 1
                cb_v[b, pl.ds(jb * L, L)] = (v & 1) << 6
            pltpu.async_copy(a_hbm.at[row_v.at[b]], gather_dst(b), gsem.at[b])

        @pl.when(t + 1 < BPW)
        def _():
            start_idx(t + 1, bo)

        @pl.when((t >= 1) & (t <= BPW))
        def _():
            @pl.when(t >= 3)
            def _():
                wait_out(bo)
            colbase = [cb_v[bo, pl.ds(jb * L, L)] for jb in range(BB // L)]
            pltpu.make_async_copy(
                a_hbm.at[row_v.at[bo]], gather_dst(bo), gsem.at[bo]).wait()

            def erow(e, c):
                for jb in range(BB // L):
                    vals = plsc.load_gather(
                        pair_v.at[bo], [jiota[jb], colbase[jb] + e])
                    tile_v[bo, e, pl.ds(jb * L, L)] = vals
                return c
            lax.fori_loop(0, EMBED, erow, 0)

            blk = blk0 + t - 1
            pltpu.async_copy(
                tile_v.at[bo],
                out_hbm.at[blk >> 7, pl.ds(0, EMBED),
                           pl.ds((blk & 127) * BB, BB)],
                wsem.at[bo])

    def pair(k, carry):
        step(2 * k, 0)
        step(2 * k + 1, 1)
        return carry

    lax.fori_loop(0, BPW // 2 + 1, pair, 0)

    wait_out(0)
    wait_out(1)


def kernel(x, A):
    out = _emb_lookup(x.T, A.reshape(VOCAB // 2, 128))
    return jnp.transpose(out, (2, 0, 1))
